# baseline (speedup 1.0000x reference)
import jax
import jax.numpy as jnp
from jax.experimental import pallas as pl
from jax.experimental.pallas import tpu as pltpu

_BLK_ROWS = 256


def _copy_body(w_ref, o_ref):
    o_ref[...] = w_ref[...]


def kernel(embedding_weight, seq_len):
    del seq_len
    S, D = embedding_weight.shape
    grid = (S // _BLK_ROWS,)
    out = pl.pallas_call(
        _copy_body,
        grid=grid,
        in_specs=[pl.BlockSpec((_BLK_ROWS, D), lambda i: (i, 0))],
        out_specs=pl.BlockSpec((_BLK_ROWS, D), lambda i: (i, 0)),
        out_shape=jax.ShapeDtypeStruct((S, D), embedding_weight.dtype),
    )(embedding_weight)
    return out[None, :, :]
